# baseline (speedup 1.0000x reference)
import jax
import jax.numpy as jnp
from jax import lax
from jax.experimental import pallas as pl
from jax.experimental.pallas import tpu as pltpu
from jax.experimental.pallas import tpu_sc as plsc

N = 10000
E = 160000
D = 256

NC = 2
NS = 16
NW = NC * NS
OWN = 320
DUMMY = OWN
ACC_ROWS = OWN + 16
CE = 2000
NQ = E // CE
NGR = CE // 16
GB = 128
PEND = 256
NPAD = N + NW * OWN - N
OUT_ROWS = NW * OWN


def _sc_body(x_ref, src_ref, dst_ref, agg_out, deg_out,
             src_v, dst_v, pend, gidx, rows_v, acc, dega, sem):
    c = lax.axis_index("c")
    s = lax.axis_index("s")
    w = s * NC + c
    lo = w * OWN

    zf = jnp.zeros((16,), jnp.float32)
    zi = jnp.zeros((16,), jnp.int32)
    oneh = (lax.iota(jnp.int32, 16) < 1).astype(jnp.int32)
    dummy_v = jnp.full((16,), DUMMY, jnp.int32)

    def zacc(r, carry):
        for k in range(D // 16):
            acc[r, pl.ds(k * 16, 16)] = zf
        return carry
    lax.fori_loop(0, ACC_ROWS, zacc, 0)
    for k in range(ACC_ROWS // 16 + 1):
        dega[pl.ds(k * 16, 16)] = zi

    def flush(base):
        for kk in range(GB // 16):
            v = pend[pl.ds(base + kk * 16, 16)]
            gidx[pl.ds(kk * 16, 16)] = v >> 9
        pltpu.async_copy(x_ref.at[gidx], rows_v, sem).wait()

        def accrow(r, carry):
            ldst = pend[pl.ds(base + r, 16)][0] & 511
            for k in range(D // 16):
                plsc.addupdate(acc.at[ldst, pl.ds(k * 16, 16)],
                               rows_v[r, pl.ds(k * 16, 16)])
            plsc.addupdate(dega.at[pl.ds(ldst, 16)], oneh)
            return carry
        lax.fori_loop(0, GB, accrow, 0)

    def chunk(q, cnt):
        eb = q * CE
        pltpu.sync_copy(src_ref.at[pl.ds(eb, CE)], src_v)
        pltpu.sync_copy(dst_ref.at[pl.ds(eb, CE)], dst_v)

        def group(i, cnt):
            vd = dst_v[pl.ds(i * 16, 16)]
            vs = src_v[pl.ds(i * 16, 16)]
            d2 = vd - lo
            m = d2.astype(jnp.uint32) < jnp.uint32(OWN)
            h = plsc.all_reduce_population_count(m)[0]

            @pl.when(h > 0)
            def _():
                packed = vs * 512 + jnp.where(m, d2, DUMMY)
                _, srt = plsc.sort_key_val(1 - m.astype(jnp.int32), packed)
                pend[pl.ds(cnt, 16)] = srt

            cnt = cnt + h

            @pl.when(cnt >= GB)
            def _():
                flush(0)
                pend[pl.ds(0, 16)] = pend[pl.ds(GB, 16)]
            cnt = jnp.where(cnt >= GB, cnt - GB, cnt)
            return cnt
        return lax.fori_loop(0, NGR, group, cnt)

    cnt = lax.fori_loop(0, NQ, chunk, jnp.int32(0))

    for k in range(GB // 16):
        pend[pl.ds(cnt + k * 16, 16)] = dummy_v

    @pl.when(cnt > 0)
    def _():
        flush(0)

    pltpu.sync_copy(acc.at[pl.ds(0, OWN)], agg_out.at[pl.ds(lo, OWN)])
    pltpu.sync_copy(dega.at[pl.ds(0, OWN)], deg_out.at[pl.ds(lo, OWN)])


_sc_aggregate = pl.kernel(
    _sc_body,
    out_type=(
        jax.ShapeDtypeStruct((OUT_ROWS, D), jnp.float32),
        jax.ShapeDtypeStruct((OUT_ROWS,), jnp.int32),
    ),
    mesh=plsc.VectorSubcoreMesh(core_axis_name="c", subcore_axis_name="s"),
    compiler_params=pltpu.CompilerParams(needs_layout_passes=False),
    scratch_types=(
        pltpu.VMEM((CE,), jnp.int32),
        pltpu.VMEM((CE,), jnp.int32),
        pltpu.VMEM((PEND,), jnp.int32),
        pltpu.VMEM((GB,), jnp.int32),
        pltpu.VMEM((GB, D), jnp.float32),
        pltpu.VMEM((ACC_ROWS, D), jnp.float32),
        pltpu.VMEM((ACC_ROWS + 16,), jnp.int32),
        pltpu.SemaphoreType.DMA,
    ),
)


BN = 400


def _tc_body(deg_ref, agg_ref, w_ref, b_ref, g_ref, be_ref, o_ref):
    d = deg_ref[...].astype(jnp.float32)
    a = agg_ref[...]
    h = a / jnp.maximum(d, 1.0)
    h = jnp.dot(h, w_ref[...], preferred_element_type=jnp.float32)
    h = h + b_ref[...]
    mu = jnp.mean(h, axis=1, keepdims=True)
    var = jnp.mean((h - mu) ** 2, axis=1, keepdims=True)
    h = (h - mu) * lax.rsqrt(var + 1e-5)
    h = h * g_ref[...] + be_ref[...]
    o_ref[...] = jnp.maximum(h, 0.0)


def _tc_dense(degp, aggp, W, b, gamma, beta):
    return pl.pallas_call(
        _tc_body,
        grid=(N // BN,),
        in_specs=[
            pl.BlockSpec((BN, 1), lambda i: (i, 0)),
            pl.BlockSpec((BN, D), lambda i: (i, 0)),
            pl.BlockSpec((D, D), lambda i: (0, 0)),
            pl.BlockSpec((1, D), lambda i: (0, 0)),
            pl.BlockSpec((1, D), lambda i: (0, 0)),
            pl.BlockSpec((1, D), lambda i: (0, 0)),
        ],
        out_specs=pl.BlockSpec((BN, D), lambda i: (i, 0)),
        out_shape=jax.ShapeDtypeStruct((N, D), jnp.float32),
    )(degp, aggp, W, b, gamma, beta)


def kernel(x, edge_index, W, b, gamma, beta):
    src = edge_index[0]
    dst = edge_index[1]
    aggp, degp = _sc_aggregate(x, src, dst)
    return _tc_dense(degp[:, None], aggp, W,
                     b[None, :], gamma[None, :], beta[None, :])


# speedup vs baseline: 1.1890x; 1.1890x over previous
import jax
import jax.numpy as jnp
from jax import lax
from jax.experimental import pallas as pl
from jax.experimental.pallas import tpu as pltpu
from jax.experimental.pallas import tpu_sc as plsc

N = 10000
E = 160000
D = 256

NC = 2
NS = 16
NW = NC * NS
OWN = 320
DUMMY = OWN
ACC_ROWS = OWN + 16
CE = 1280
NQ = E // CE
NGR = CE // 16
NB = NGR // 8
GB = 128
PEND = 256
OUT_ROWS = NW * OWN


def _sc_body(x_ref, src_ref, dst_ref, agg_out, deg_out,
             src_v, dst_v, pend, gidx, rows_v, acc, dega, sem):
    c = lax.axis_index("c")
    s = lax.axis_index("s")
    w = s * NC + c
    lo = w * OWN

    zf = jnp.zeros((16,), jnp.float32)
    zi = jnp.zeros((16,), jnp.int32)
    oneh = (lax.iota(jnp.int32, 16) < 1).astype(jnp.int32)
    dummy_v = jnp.full((16,), DUMMY, jnp.int32)

    def zacc(r, carry):
        for k in range(D // 16):
            acc[r, pl.ds(k * 16, 16)] = zf
        return carry
    lax.fori_loop(0, ACC_ROWS, zacc, 0)
    for k in range(ACC_ROWS // 16 + 1):
        dega[pl.ds(k * 16, 16)] = zi

    def flush(base):
        for kk in range(GB // 16):
            v = pend[pl.ds(base + kk * 16, 16)]
            gidx[pl.ds(kk * 16, 16)] = v >> 9
        pltpu.async_copy(x_ref.at[gidx], rows_v, sem).wait()

        def accrow(r, carry):
            ldst = pend[pl.ds(base + r, 16)][0] & 511
            for k in range(D // 16):
                plsc.addupdate(acc.at[ldst, pl.ds(k * 16, 16)],
                               rows_v[r, pl.ds(k * 16, 16)])
            plsc.addupdate(dega.at[pl.ds(ldst, 16)], oneh)
            return carry
        lax.fori_loop(0, GB, accrow, 0)

    def chunk(q, cnt):
        eb = q * CE
        pltpu.sync_copy(src_ref.at[pl.ds(eb, CE)], src_v)
        pltpu.sync_copy(dst_ref.at[pl.ds(eb, CE)], dst_v)

        def block(bi, cnt):
            for g in range(8):
                i = bi * 8 + g
                vd = dst_v[pl.ds(i * 16, 16)]
                vs = src_v[pl.ds(i * 16, 16)]
                d2 = vd - lo
                m = d2.astype(jnp.uint32) < jnp.uint32(OWN)
                h = plsc.all_reduce_population_count(m)[0]
                packed = vs * 512 + jnp.where(m, d2, DUMMY)
                _, srt = plsc.sort_key_val(1 - m.astype(jnp.int32), packed)
                pend[pl.ds(cnt, 16)] = srt
                cnt = cnt + h

            @pl.when(cnt >= GB)
            def _():
                flush(0)
                for k in range(8):
                    pend[pl.ds(k * 16, 16)] = pend[pl.ds(GB + k * 16, 16)]
            cnt = jnp.where(cnt >= GB, cnt - GB, cnt)
            return cnt
        return lax.fori_loop(0, NB, block, cnt)

    cnt = lax.fori_loop(0, NQ, chunk, jnp.int32(0))

    for k in range(GB // 16):
        pend[pl.ds(cnt + k * 16, 16)] = dummy_v

    @pl.when(cnt > 0)
    def _():
        flush(0)

    pltpu.sync_copy(acc.at[pl.ds(0, OWN)], agg_out.at[pl.ds(lo, OWN)])
    pltpu.sync_copy(dega.at[pl.ds(0, OWN)], deg_out.at[pl.ds(lo, OWN)])


_sc_aggregate = pl.kernel(
    _sc_body,
    out_type=(
        jax.ShapeDtypeStruct((OUT_ROWS, D), jnp.float32),
        jax.ShapeDtypeStruct((OUT_ROWS,), jnp.int32),
    ),
    mesh=plsc.VectorSubcoreMesh(core_axis_name="c", subcore_axis_name="s"),
    compiler_params=pltpu.CompilerParams(needs_layout_passes=False),
    scratch_types=(
        pltpu.VMEM((CE,), jnp.int32),
        pltpu.VMEM((CE,), jnp.int32),
        pltpu.VMEM((PEND,), jnp.int32),
        pltpu.VMEM((GB,), jnp.int32),
        pltpu.VMEM((GB, D), jnp.float32),
        pltpu.VMEM((ACC_ROWS, D), jnp.float32),
        pltpu.VMEM((ACC_ROWS + 16,), jnp.int32),
        pltpu.SemaphoreType.DMA,
    ),
)


BN = 400


def _tc_body(deg_ref, agg_ref, w_ref, b_ref, g_ref, be_ref, o_ref):
    d = deg_ref[...].astype(jnp.float32)
    a = agg_ref[...]
    h = a / jnp.maximum(d, 1.0)
    h = jnp.dot(h, w_ref[...], preferred_element_type=jnp.float32)
    h = h + b_ref[...]
    mu = jnp.mean(h, axis=1, keepdims=True)
    var = jnp.mean((h - mu) ** 2, axis=1, keepdims=True)
    h = (h - mu) * lax.rsqrt(var + 1e-5)
    h = h * g_ref[...] + be_ref[...]
    o_ref[...] = jnp.maximum(h, 0.0)


def _tc_dense(degp, aggp, W, b, gamma, beta):
    return pl.pallas_call(
        _tc_body,
        grid=(N // BN,),
        in_specs=[
            pl.BlockSpec((BN, 1), lambda i: (i, 0)),
            pl.BlockSpec((BN, D), lambda i: (i, 0)),
            pl.BlockSpec((D, D), lambda i: (0, 0)),
            pl.BlockSpec((1, D), lambda i: (0, 0)),
            pl.BlockSpec((1, D), lambda i: (0, 0)),
            pl.BlockSpec((1, D), lambda i: (0, 0)),
        ],
        out_specs=pl.BlockSpec((BN, D), lambda i: (i, 0)),
        out_shape=jax.ShapeDtypeStruct((N, D), jnp.float32),
    )(degp, aggp, W, b, gamma, beta)


def kernel(x, edge_index, W, b, gamma, beta):
    src = edge_index[0]
    dst = edge_index[1]
    aggp, degp = _sc_aggregate(x, src, dst)
    return _tc_dense(degp[:, None], aggp, W,
                     b[None, :], gamma[None, :], beta[None, :])


# speedup vs baseline: 1.5850x; 1.3330x over previous
import jax
import jax.numpy as jnp
from jax import lax
from jax.experimental import pallas as pl
from jax.experimental.pallas import tpu as pltpu
from jax.experimental.pallas import tpu_sc as plsc

N = 10000
E = 160000
D = 256

NC = 2
NS = 16
NW = NC * NS
OWN = 320
DUMMY = OWN
ACC_ROWS = OWN + 1
CE = 6400
NQ = E // CE
NGR = CE // 16
NB = NGR // 8
GB = 128
PEND = 256
OUT_ROWS = NW * OWN


def _sc_body(x_ref, src_ref, dst_ref, agg_out, deg_out,
             src_v, dst_v, pend, gidx, rows_v, acc, dega, sem):
    c = lax.axis_index("c")
    s = lax.axis_index("s")
    w = s * NC + c
    lo = w * OWN

    zf = jnp.zeros((16,), jnp.float32)
    zi = jnp.zeros((16,), jnp.int32)
    oneh = (lax.iota(jnp.int32, 16) < 1).astype(jnp.int32)
    dummy_v = jnp.full((16,), DUMMY * 16384, jnp.int32)

    def zacc(r, carry):
        for k in range(D // 16):
            acc[r, pl.ds(k * 16, 16)] = zf
        return carry
    lax.fori_loop(0, ACC_ROWS, zacc, 0)
    for k in range(352 // 16):
        dega[pl.ds(k * 16, 16)] = zi

    def flush(base):
        for kk in range(GB // 16):
            v = pend[pl.ds(base + kk * 16, 16)]
            gidx[pl.ds(kk * 16, 16)] = v & 16383
        pltpu.async_copy(x_ref.at[gidx], rows_v, sem).wait()

        def accrow(r, carry):
            ldst = pend[pl.ds(base + r, 16)][0] >> 14
            for k in range(D // 16):
                plsc.addupdate(acc.at[ldst, pl.ds(k * 16, 16)],
                               rows_v[r, pl.ds(k * 16, 16)])
            plsc.addupdate(dega.at[pl.ds(ldst, 16)], oneh)
            return carry
        lax.fori_loop(0, GB, accrow, 0)

    def chunk(q, cnt):
        eb = q * CE
        pltpu.sync_copy(src_ref.at[pl.ds(eb, CE)], src_v)
        pltpu.sync_copy(dst_ref.at[pl.ds(eb, CE)], dst_v)

        def block(bi, cnt):
            srts, hs = [], []
            for g in range(8):
                i = bi * 8 + g
                vd = dst_v[pl.ds(i * 16, 16)]
                vs = src_v[pl.ds(i * 16, 16)]
                d2 = vd - lo
                d2u = d2.astype(jnp.uint32)
                h = plsc.all_reduce_population_count(d2u < jnp.uint32(OWN))[0]
                ldst = jnp.minimum(d2u, jnp.uint32(DUMMY)).astype(jnp.int32)
                packed = ldst * 16384 + vs
                _, srt = plsc.sort_key_val(packed, packed)
                srts.append(srt)
                hs.append(h)
            for g in range(8):
                pend[pl.ds(cnt, 16)] = srts[g]
                cnt = cnt + hs[g]

            @pl.when(cnt >= GB)
            def _():
                flush(0)
                for k in range(8):
                    pend[pl.ds(k * 16, 16)] = pend[pl.ds(GB + k * 16, 16)]
            cnt = jnp.where(cnt >= GB, cnt - GB, cnt)
            return cnt
        return lax.fori_loop(0, NB, block, cnt)

    cnt = lax.fori_loop(0, NQ, chunk, jnp.int32(0))

    for k in range(GB // 16):
        pend[pl.ds(cnt + k * 16, 16)] = dummy_v

    @pl.when(cnt > 0)
    def _():
        flush(0)

    pltpu.sync_copy(acc.at[pl.ds(0, OWN)], agg_out.at[pl.ds(lo, OWN)])
    pltpu.sync_copy(dega.at[pl.ds(0, OWN)], deg_out.at[pl.ds(lo, OWN)])


_sc_aggregate = pl.kernel(
    _sc_body,
    out_type=(
        jax.ShapeDtypeStruct((OUT_ROWS, D), jnp.float32),
        jax.ShapeDtypeStruct((OUT_ROWS,), jnp.int32),
    ),
    mesh=plsc.VectorSubcoreMesh(core_axis_name="c", subcore_axis_name="s"),
    compiler_params=pltpu.CompilerParams(needs_layout_passes=False),
    scratch_types=(
        pltpu.VMEM((CE,), jnp.int32),
        pltpu.VMEM((CE,), jnp.int32),
        pltpu.VMEM((PEND,), jnp.int32),
        pltpu.VMEM((GB,), jnp.int32),
        pltpu.VMEM((GB, D), jnp.float32),
        pltpu.VMEM((ACC_ROWS, D), jnp.float32),
        pltpu.VMEM((352,), jnp.int32),
        pltpu.SemaphoreType.DMA,
    ),
)


BN = 400


def _tc_body(deg_ref, agg_ref, w_ref, b_ref, g_ref, be_ref, o_ref):
    d = deg_ref[...].astype(jnp.float32)
    a = agg_ref[...]
    h = a / jnp.maximum(d, 1.0)
    h = jnp.dot(h, w_ref[...], preferred_element_type=jnp.float32)
    h = h + b_ref[...]
    mu = jnp.mean(h, axis=1, keepdims=True)
    var = jnp.mean((h - mu) ** 2, axis=1, keepdims=True)
    h = (h - mu) * lax.rsqrt(var + 1e-5)
    h = h * g_ref[...] + be_ref[...]
    o_ref[...] = jnp.maximum(h, 0.0)


def _tc_dense(degp, aggp, W, b, gamma, beta):
    return pl.pallas_call(
        _tc_body,
        grid=(N // BN,),
        in_specs=[
            pl.BlockSpec((BN, 1), lambda i: (i, 0)),
            pl.BlockSpec((BN, D), lambda i: (i, 0)),
            pl.BlockSpec((D, D), lambda i: (0, 0)),
            pl.BlockSpec((1, D), lambda i: (0, 0)),
            pl.BlockSpec((1, D), lambda i: (0, 0)),
            pl.BlockSpec((1, D), lambda i: (0, 0)),
        ],
        out_specs=pl.BlockSpec((BN, D), lambda i: (i, 0)),
        out_shape=jax.ShapeDtypeStruct((N, D), jnp.float32),
    )(degp, aggp, W, b, gamma, beta)


def kernel(x, edge_index, W, b, gamma, beta):
    src = edge_index[0]
    dst = edge_index[1]
    aggp, degp = _sc_aggregate(x, src, dst)
    return _tc_dense(degp[:, None], aggp, W,
                     b[None, :], gamma[None, :], beta[None, :])


# speedup vs baseline: 1.6804x; 1.0602x over previous
import jax
import jax.numpy as jnp
from jax import lax
from jax.experimental import pallas as pl
from jax.experimental.pallas import tpu as pltpu
from jax.experimental.pallas import tpu_sc as plsc

N = 10000
E = 160000
D = 256

NC = 2
NS = 16
NW = NC * NS
OWN = 320
DUMMY = OWN
ACC_ROWS = OWN + 1
CE = 3200
NQ = E // CE
NP = NQ // 2
NGR = CE // 16
NB = NGR // 8
GB = 128
PEND = 256
OUT_ROWS = NW * OWN


def _sc_body(x_ref, src_ref, dst_ref, agg_out, deg_out,
             src0, dst0, src1, dst1, pend, gidx, rows_v, acc, dega,
             sem, semA, semB):
    c = lax.axis_index("c")
    s = lax.axis_index("s")
    w = s * NC + c
    lo = w * OWN

    zf = jnp.zeros((16,), jnp.float32)
    zi = jnp.zeros((16,), jnp.int32)
    oneh = (lax.iota(jnp.int32, 16) < 1).astype(jnp.int32)
    dummy_v = jnp.full((16,), DUMMY * 16384, jnp.int32)

    def zacc(r, carry):
        for k in range(D // 16):
            acc[r, pl.ds(k * 16, 16)] = zf
        return carry
    lax.fori_loop(0, ACC_ROWS, zacc, 0)
    for k in range(352 // 16):
        dega[pl.ds(k * 16, 16)] = zi

    def flush(base):
        for kk in range(GB // 16):
            v = pend[pl.ds(base + kk * 16, 16)]
            gidx[pl.ds(kk * 16, 16)] = v & 16383
        pltpu.async_copy(x_ref.at[gidx], rows_v, sem).wait()

        def accrow(r, carry):
            ldst = pend[pl.ds(base + r, 16)][0] >> 14
            for k in range(D // 16):
                plsc.addupdate(acc.at[ldst, pl.ds(k * 16, 16)],
                               rows_v[r, pl.ds(k * 16, 16)])
            plsc.addupdate(dega.at[pl.ds(ldst, 16)], oneh)
            return carry
        lax.fori_loop(0, GB, accrow, 0)

    def load(q, sbuf, dbuf, sm):
        pltpu.async_copy(src_ref.at[pl.ds(q * CE, CE)], sbuf, sm)
        pltpu.async_copy(dst_ref.at[pl.ds(q * CE, CE)], dbuf, sm)

    def loadwait(q, sbuf, dbuf, sm):
        pltpu.make_async_copy(src_ref.at[pl.ds(q * CE, CE)], sbuf, sm).wait()
        pltpu.make_async_copy(dst_ref.at[pl.ds(q * CE, CE)], dbuf, sm).wait()

    def scan_buffer(src_v, dst_v, cnt):
        def block(bi, cnt):
            srts, hs = [], []
            for g in range(8):
                i = bi * 8 + g
                vd = dst_v[pl.ds(i * 16, 16)]
                vs = src_v[pl.ds(i * 16, 16)]
                d2 = vd - lo
                d2u = d2.astype(jnp.uint32)
                h = plsc.all_reduce_population_count(d2u < jnp.uint32(OWN))[0]
                ldst = jnp.minimum(d2u, jnp.uint32(DUMMY)).astype(jnp.int32)
                packed = ldst * 16384 + vs
                _, srt = plsc.sort_key_val(packed, packed)
                srts.append(srt)
                hs.append(h)
            for g in range(8):
                pend[pl.ds(cnt, 16)] = srts[g]
                cnt = cnt + hs[g]

            @pl.when(cnt >= GB)
            def _():
                flush(0)
                for k in range(8):
                    pend[pl.ds(k * 16, 16)] = pend[pl.ds(GB + k * 16, 16)]
            cnt = jnp.where(cnt >= GB, cnt - GB, cnt)
            return cnt
        return lax.fori_loop(0, NB, block, cnt)

    load(0, src0, dst0, semA)

    def pair(p, cnt):
        q0 = 2 * p
        loadwait(q0, src0, dst0, semA)
        load(q0 + 1, src1, dst1, semB)
        cnt = scan_buffer(src0, dst0, cnt)
        loadwait(q0 + 1, src1, dst1, semB)

        @pl.when(p < NP - 1)
        def _():
            load(q0 + 2, src0, dst0, semA)
        cnt = scan_buffer(src1, dst1, cnt)
        return cnt

    cnt = lax.fori_loop(0, NP, pair, jnp.int32(0))

    for k in range(GB // 16):
        pend[pl.ds(cnt + k * 16, 16)] = dummy_v

    @pl.when(cnt > 0)
    def _():
        flush(0)

    pltpu.sync_copy(acc.at[pl.ds(0, OWN)], agg_out.at[pl.ds(lo, OWN)])
    pltpu.sync_copy(dega.at[pl.ds(0, OWN)], deg_out.at[pl.ds(lo, OWN)])


_sc_aggregate = pl.kernel(
    _sc_body,
    out_type=(
        jax.ShapeDtypeStruct((OUT_ROWS, D), jnp.float32),
        jax.ShapeDtypeStruct((OUT_ROWS,), jnp.int32),
    ),
    mesh=plsc.VectorSubcoreMesh(core_axis_name="c", subcore_axis_name="s"),
    compiler_params=pltpu.CompilerParams(needs_layout_passes=False),
    scratch_types=(
        pltpu.VMEM((CE,), jnp.int32),
        pltpu.VMEM((CE,), jnp.int32),
        pltpu.VMEM((CE,), jnp.int32),
        pltpu.VMEM((CE,), jnp.int32),
        pltpu.VMEM((PEND,), jnp.int32),
        pltpu.VMEM((GB,), jnp.int32),
        pltpu.VMEM((GB, D), jnp.float32),
        pltpu.VMEM((ACC_ROWS, D), jnp.float32),
        pltpu.VMEM((352,), jnp.int32),
        pltpu.SemaphoreType.DMA,
        pltpu.SemaphoreType.DMA,
        pltpu.SemaphoreType.DMA,
    ),
)


BN = 400


def _tc_body(deg_ref, agg_ref, w_ref, b_ref, g_ref, be_ref, o_ref):
    d = deg_ref[...].astype(jnp.float32)
    a = agg_ref[...]
    h = a / jnp.maximum(d, 1.0)
    h = jnp.dot(h, w_ref[...], preferred_element_type=jnp.float32)
    h = h + b_ref[...]
    mu = jnp.mean(h, axis=1, keepdims=True)
    var = jnp.mean((h - mu) ** 2, axis=1, keepdims=True)
    h = (h - mu) * lax.rsqrt(var + 1e-5)
    h = h * g_ref[...] + be_ref[...]
    o_ref[...] = jnp.maximum(h, 0.0)


def _tc_dense(degp, aggp, W, b, gamma, beta):
    return pl.pallas_call(
        _tc_body,
        grid=(N // BN,),
        in_specs=[
            pl.BlockSpec((BN, 1), lambda i: (i, 0)),
            pl.BlockSpec((BN, D), lambda i: (i, 0)),
            pl.BlockSpec((D, D), lambda i: (0, 0)),
            pl.BlockSpec((1, D), lambda i: (0, 0)),
            pl.BlockSpec((1, D), lambda i: (0, 0)),
            pl.BlockSpec((1, D), lambda i: (0, 0)),
        ],
        out_specs=pl.BlockSpec((BN, D), lambda i: (i, 0)),
        out_shape=jax.ShapeDtypeStruct((N, D), jnp.float32),
    )(degp, aggp, W, b, gamma, beta)


def kernel(x, edge_index, W, b, gamma, beta):
    src = edge_index[0]
    dst = edge_index[1]
    aggp, degp = _sc_aggregate(x, src, dst)
    return _tc_dense(degp[:, None], aggp, W,
                     b[None, :], gamma[None, :], beta[None, :])


# speedup vs baseline: 1.7507x; 1.0418x over previous
import jax
import jax.numpy as jnp
from jax import lax
from jax.experimental import pallas as pl
from jax.experimental.pallas import tpu as pltpu
from jax.experimental.pallas import tpu_sc as plsc

N = 10000
E = 160000
D = 256

NC = 2
NS = 16
NW = NC * NS
OWN = 320
DUMMY = OWN
ACC_ROWS = OWN + 1
CE = 3200
NQ = E // CE
NP = NQ // 2
NGR = CE // 16
NB = NGR // 16
GB = 128
PEND = 416
OUT_ROWS = NW * OWN


def _sc_body(x_ref, src_ref, dst_ref, agg_out, deg_out,
             src0, dst0, src1, dst1, pend, gidx, rows_v, acc, dega,
             sem, semA, semB):
    c = lax.axis_index("c")
    s = lax.axis_index("s")
    w = s * NC + c
    lo = w * OWN

    zf = jnp.zeros((16,), jnp.float32)
    zi = jnp.zeros((16,), jnp.int32)
    oneh = (lax.iota(jnp.int32, 16) < 1).astype(jnp.int32)
    dummy_v = jnp.full((16,), DUMMY * 16384, jnp.int32)

    def zacc(r, carry):
        for k in range(D // 16):
            acc[r, pl.ds(k * 16, 16)] = zf
        return carry
    lax.fori_loop(0, ACC_ROWS, zacc, 0)
    for k in range(352 // 16):
        dega[pl.ds(k * 16, 16)] = zi

    def flush(base):
        for kk in range(GB // 16):
            v = pend[pl.ds(base + kk * 16, 16)]
            gidx[pl.ds(kk * 16, 16)] = v & 16383
        pltpu.async_copy(x_ref.at[gidx], rows_v, sem).wait()

        def accrow(r, carry):
            ldst = pend[pl.ds(base + r, 16)][0] >> 14
            for k in range(D // 16):
                plsc.addupdate(acc.at[ldst, pl.ds(k * 16, 16)],
                               rows_v[r, pl.ds(k * 16, 16)])
            plsc.addupdate(dega.at[pl.ds(ldst, 16)], oneh)
            return carry
        lax.fori_loop(0, GB, accrow, 0)

    def load(q, sbuf, dbuf, sm):
        pltpu.async_copy(src_ref.at[pl.ds(q * CE, CE)], sbuf, sm)
        pltpu.async_copy(dst_ref.at[pl.ds(q * CE, CE)], dbuf, sm)

    def loadwait(q, sbuf, dbuf, sm):
        pltpu.make_async_copy(src_ref.at[pl.ds(q * CE, CE)], sbuf, sm).wait()
        pltpu.make_async_copy(dst_ref.at[pl.ds(q * CE, CE)], dbuf, sm).wait()

    def scan_buffer(src_v, dst_v, cnt):
        def block(bi, cnt):
            srts, hs = [], []
            for g in range(16):
                i = bi * 16 + g
                vd = dst_v[pl.ds(i * 16, 16)]
                vs = src_v[pl.ds(i * 16, 16)]
                d2 = vd - lo
                d2u = d2.astype(jnp.uint32)
                h = plsc.all_reduce_population_count(d2u < jnp.uint32(OWN))[0]
                ldst = jnp.minimum(d2u, jnp.uint32(DUMMY)).astype(jnp.int32)
                packed = ldst * 16384 + vs
                _, srt = plsc.sort_key_val(packed, packed)
                srts.append(srt)
                hs.append(h)
            for g in range(16):
                pend[pl.ds(cnt, 16)] = srts[g]
                cnt = cnt + hs[g]

            big = cnt >= 2 * GB

            @pl.when(big)
            def _():
                flush(0)
                flush(GB)
                for k in range(8):
                    pend[pl.ds(k * 16, 16)] = pend[pl.ds(2 * GB + k * 16, 16)]
            cnt = jnp.where(big, cnt - 2 * GB, cnt)
            mid = cnt >= GB

            @pl.when(mid)
            def _():
                flush(0)
                for k in range(8):
                    pend[pl.ds(k * 16, 16)] = pend[pl.ds(GB + k * 16, 16)]
            cnt = jnp.where(mid, cnt - GB, cnt)
            return cnt
        return lax.fori_loop(0, NB, block, cnt)

    load(0, src0, dst0, semA)

    def pair(p, cnt):
        q0 = 2 * p
        loadwait(q0, src0, dst0, semA)
        load(q0 + 1, src1, dst1, semB)
        cnt = scan_buffer(src0, dst0, cnt)
        loadwait(q0 + 1, src1, dst1, semB)

        @pl.when(p < NP - 1)
        def _():
            load(q0 + 2, src0, dst0, semA)
        cnt = scan_buffer(src1, dst1, cnt)
        return cnt

    cnt = lax.fori_loop(0, NP, pair, jnp.int32(0))

    for k in range(GB // 16):
        pend[pl.ds(cnt + k * 16, 16)] = dummy_v

    @pl.when(cnt > 0)
    def _():
        flush(0)

    pltpu.sync_copy(acc.at[pl.ds(0, OWN)], agg_out.at[pl.ds(lo, OWN)])
    pltpu.sync_copy(dega.at[pl.ds(0, OWN)], deg_out.at[pl.ds(lo, OWN)])


_sc_aggregate = pl.kernel(
    _sc_body,
    out_type=(
        jax.ShapeDtypeStruct((OUT_ROWS, D), jnp.float32),
        jax.ShapeDtypeStruct((OUT_ROWS,), jnp.int32),
    ),
    mesh=plsc.VectorSubcoreMesh(core_axis_name="c", subcore_axis_name="s"),
    compiler_params=pltpu.CompilerParams(needs_layout_passes=False),
    scratch_types=(
        pltpu.VMEM((CE,), jnp.int32),
        pltpu.VMEM((CE,), jnp.int32),
        pltpu.VMEM((CE,), jnp.int32),
        pltpu.VMEM((CE,), jnp.int32),
        pltpu.VMEM((PEND,), jnp.int32),
        pltpu.VMEM((GB,), jnp.int32),
        pltpu.VMEM((GB, D), jnp.float32),
        pltpu.VMEM((ACC_ROWS, D), jnp.float32),
        pltpu.VMEM((352,), jnp.int32),
        pltpu.SemaphoreType.DMA,
        pltpu.SemaphoreType.DMA,
        pltpu.SemaphoreType.DMA,
    ),
)


BN = 400


def _tc_body(deg_ref, agg_ref, w_ref, b_ref, g_ref, be_ref, o_ref):
    d = deg_ref[...].astype(jnp.float32)
    a = agg_ref[...]
    h = a / jnp.maximum(d, 1.0)
    h = jnp.dot(h, w_ref[...], preferred_element_type=jnp.float32)
    h = h + b_ref[...]
    mu = jnp.mean(h, axis=1, keepdims=True)
    var = jnp.mean((h - mu) ** 2, axis=1, keepdims=True)
    h = (h - mu) * lax.rsqrt(var + 1e-5)
    h = h * g_ref[...] + be_ref[...]
    o_ref[...] = jnp.maximum(h, 0.0)


def _tc_dense(degp, aggp, W, b, gamma, beta):
    return pl.pallas_call(
        _tc_body,
        grid=(N // BN,),
        in_specs=[
            pl.BlockSpec((BN, 1), lambda i: (i, 0)),
            pl.BlockSpec((BN, D), lambda i: (i, 0)),
            pl.BlockSpec((D, D), lambda i: (0, 0)),
            pl.BlockSpec((1, D), lambda i: (0, 0)),
            pl.BlockSpec((1, D), lambda i: (0, 0)),
            pl.BlockSpec((1, D), lambda i: (0, 0)),
        ],
        out_specs=pl.BlockSpec((BN, D), lambda i: (i, 0)),
        out_shape=jax.ShapeDtypeStruct((N, D), jnp.float32),
    )(degp, aggp, W, b, gamma, beta)


def kernel(x, edge_index, W, b, gamma, beta):
    src = edge_index[0]
    dst = edge_index[1]
    aggp, degp = _sc_aggregate(x, src, dst)
    return _tc_dense(degp[:, None], aggp, W,
                     b[None, :], gamma[None, :], beta[None, :])
